# baseline (speedup 1.0000x reference)
import jax
import jax.numpy as jnp
from jax.experimental import pallas as pl
from jax.experimental.pallas import tpu as pltpu

_NCHUNK = 8
_DEPTH = 2


def _film_body(task_ref, x_hbm, g_ref, b_ref, o_hbm, inbuf, outbuf,
               insem, outsem):
    del task_ref
    nch, ch, _ = inbuf.shape
    g = g_ref[0]
    b = b_ref[0]

    def read(c):
        return pltpu.make_async_copy(
            x_hbm.at[pl.ds(c * ch, ch), :], inbuf.at[c], insem.at[c])

    def write(c):
        return pltpu.make_async_copy(
            outbuf.at[c], o_hbm.at[pl.ds(c * ch, ch), :], outsem.at[c])

    for c in range(_DEPTH):
        read(c).start()
    for c in range(nch):
        read(c).wait()
        outbuf[c] = inbuf[c] * g + b
        write(c).start()
        if c + _DEPTH < nch:
            read(c + _DEPTH).start()
    for c in range(nch):
        write(c).wait()


def kernel(x, gamma_w, beta_w, task_id):
    batch, dim = x.shape
    num_tasks = gamma_w.shape[0]
    task = jnp.asarray(task_id, dtype=jnp.int32).reshape((1,))
    g3 = gamma_w.reshape(num_tasks, 1, dim)
    b3 = beta_w.reshape(num_tasks, 1, dim)
    nch = _NCHUNK
    ch = batch // nch
    return pl.pallas_call(
        _film_body,
        grid_spec=pltpu.PrefetchScalarGridSpec(
            num_scalar_prefetch=1,
            grid=(1,),
            in_specs=[
                pl.BlockSpec(memory_space=pl.ANY),
                pl.BlockSpec((1, 1, dim), lambda i, t: (t[0], 0, 0)),
                pl.BlockSpec((1, 1, dim), lambda i, t: (t[0], 0, 0)),
            ],
            out_specs=pl.BlockSpec(memory_space=pl.ANY),
            scratch_shapes=[
                pltpu.VMEM((nch, ch, dim), jnp.float32),
                pltpu.VMEM((nch, ch, dim), jnp.float32),
                pltpu.SemaphoreType.DMA((nch,)),
                pltpu.SemaphoreType.DMA((nch,)),
            ],
        ),
        out_shape=jax.ShapeDtypeStruct(x.shape, x.dtype),
    )(task, x, g3, b3)


# speedup vs baseline: 1.3848x; 1.3848x over previous
import jax
import jax.numpy as jnp
from jax.experimental import pallas as pl
from jax.experimental.pallas import tpu as pltpu

_NCHUNK = 2
_DEPTH = 2


def _film_body(task_ref, x_hbm, g_ref, b_ref, o_hbm, inbuf, outbuf,
               insem, outsem):
    del task_ref
    nch, ch, _ = inbuf.shape
    g = g_ref[0]
    b = b_ref[0]

    def read(c):
        return pltpu.make_async_copy(
            x_hbm.at[pl.ds(c * ch, ch), :], inbuf.at[c], insem.at[c])

    def write(c):
        return pltpu.make_async_copy(
            outbuf.at[c], o_hbm.at[pl.ds(c * ch, ch), :], outsem.at[c])

    for c in range(_DEPTH):
        read(c).start()
    for c in range(nch):
        read(c).wait()
        outbuf[c] = inbuf[c] * g + b
        write(c).start()
        if c + _DEPTH < nch:
            read(c + _DEPTH).start()
    for c in range(nch):
        write(c).wait()


def kernel(x, gamma_w, beta_w, task_id):
    batch, dim = x.shape
    num_tasks = gamma_w.shape[0]
    task = jnp.asarray(task_id, dtype=jnp.int32).reshape((1,))
    g3 = gamma_w.reshape(num_tasks, 1, dim)
    b3 = beta_w.reshape(num_tasks, 1, dim)
    nch = _NCHUNK
    ch = batch // nch
    return pl.pallas_call(
        _film_body,
        grid_spec=pltpu.PrefetchScalarGridSpec(
            num_scalar_prefetch=1,
            grid=(1,),
            in_specs=[
                pl.BlockSpec(memory_space=pl.ANY),
                pl.BlockSpec((1, 1, dim), lambda i, t: (t[0], 0, 0)),
                pl.BlockSpec((1, 1, dim), lambda i, t: (t[0], 0, 0)),
            ],
            out_specs=pl.BlockSpec(memory_space=pl.ANY),
            scratch_shapes=[
                pltpu.VMEM((nch, ch, dim), jnp.float32),
                pltpu.VMEM((nch, ch, dim), jnp.float32),
                pltpu.SemaphoreType.DMA((nch,)),
                pltpu.SemaphoreType.DMA((nch,)),
            ],
        ),
        out_shape=jax.ShapeDtypeStruct(x.shape, x.dtype),
    )(task, x, g3, b3)


# speedup vs baseline: 1.5420x; 1.1135x over previous
import jax
import jax.numpy as jnp
from jax.experimental import pallas as pl
from jax.experimental.pallas import tpu as pltpu

_BLOCK_B = 8192


def _film_body(task_ref, x_ref, g_ref, b_ref, o_ref):
    del task_ref
    o_ref[...] = x_ref[...] * g_ref[0] + b_ref[0]


def kernel(x, gamma_w, beta_w, task_id):
    batch, dim = x.shape
    num_tasks = gamma_w.shape[0]
    task = jnp.asarray(task_id, dtype=jnp.int32).reshape((1,))
    g3 = gamma_w.reshape(num_tasks, 1, dim)
    b3 = beta_w.reshape(num_tasks, 1, dim)
    block_b = min(_BLOCK_B, batch)
    grid = (batch // block_b,)
    return pl.pallas_call(
        _film_body,
        grid_spec=pltpu.PrefetchScalarGridSpec(
            num_scalar_prefetch=1,
            grid=grid,
            in_specs=[
                pl.BlockSpec((block_b, dim), lambda i, t: (i, 0)),
                pl.BlockSpec((1, 1, dim), lambda i, t: (t[0], 0, 0)),
                pl.BlockSpec((1, 1, dim), lambda i, t: (t[0], 0, 0)),
            ],
            out_specs=pl.BlockSpec((block_b, dim), lambda i, t: (i, 0)),
        ),
        out_shape=jax.ShapeDtypeStruct(x.shape, x.dtype),
        compiler_params=pltpu.CompilerParams(
            dimension_semantics=("parallel",),
        ),
    )(task, x, g3, b3)
